# baseline (speedup 1.0000x reference)
import functools

import jax
import jax.numpy as jnp
from jax import lax
from jax.experimental import pallas as pl
from jax.experimental.pallas import tpu as pltpu
from jax.experimental.pallas import tpu_sc as plsc

_B, _C, _H, _W = 8, 64, 32, 32
_K = 8192
_N = _B * _H * _W
_BETA = 0.25

_R = 512
_KB = _K // 2
_NR = _N // _R


def _dist_argmin_body(zf_ref, cb_ref, zsq_ref, csq_ref,
                      idx_ref, msum_ref, v0_ref, i0_ref, acc_ref):
    i = pl.program_id(0)
    k = pl.program_id(1)
    mm = lax.dot_general(zf_ref[...], cb_ref[...],
                         (((1,), (1,)), ((), ())),
                         preferred_element_type=jnp.float32)
    d = (zsq_ref[...] + csq_ref[...]) - 2.0 * mm
    lmin = jnp.min(d, axis=1, keepdims=True)
    idsf = lax.broadcasted_iota(jnp.int32, (_R, _KB), 1).astype(jnp.float32)
    lidxf = jnp.min(jnp.where(d == lmin, idsf, float(_K)),
                    axis=1, keepdims=True)

    @pl.when(k == 0)
    def _():
        v0_ref[...] = lmin
        i0_ref[...] = lidxf

    @pl.when(k == 1)
    def _():
        v0 = v0_ref[...]
        v0r = v0.astype(jnp.bfloat16).astype(jnp.float32)
        take = lmin < v0r
        idxf = jnp.where(take, lidxf + float(_KB), i0_ref[...])
        idx_ref[...] = idxf.astype(jnp.int32)
        vsel = jnp.where(take, lmin, v0)
        prev = jnp.where(i == 0, 0.0, acc_ref[0])
        acc_ref[0] = prev + jnp.sum(vsel)

        @pl.when(i == _NR - 1)
        def _():
            msum_ref[0] = acc_ref[0]


_dist_argmin = pl.pallas_call(
    _dist_argmin_body,
    grid=(_NR, 2),
    in_specs=[
        pl.BlockSpec((_R, _C), lambda i, k: (i, 0)),
        pl.BlockSpec((_KB, _C), lambda i, k: (k, 0)),
        pl.BlockSpec((_R, 1), lambda i, k: (i, 0)),
        pl.BlockSpec((1, _KB), lambda i, k: (0, k)),
    ],
    out_specs=(
        pl.BlockSpec((_R, 1), lambda i, k: (i, 0)),
        pl.BlockSpec(memory_space=pltpu.SMEM),
    ),
    out_shape=(
        jax.ShapeDtypeStruct((_N, 1), jnp.int32),
        jax.ShapeDtypeStruct((1,), jnp.float32),
    ),
    scratch_shapes=[
        pltpu.VMEM((_R, 1), jnp.float32),
        pltpu.VMEM((_R, 1), jnp.float32),
        pltpu.SMEM((1,), jnp.float32),
    ],
)


_SC_INFO = plsc.get_sparse_core_info()
_NTILES = _SC_INFO.num_cores * _SC_INFO.num_subcores
_BPW = _N // _NTILES
_CHUNK = 128
_NCH = _BPW // _CHUNK
_CP = _C


def _make_sc_gather():
    mesh = plsc.VectorSubcoreMesh(core_axis_name="c", subcore_axis_name="s")

    @functools.partial(
        pl.kernel, mesh=mesh,
        out_type=jax.ShapeDtypeStruct((_N, _CP), jnp.float32),
        scratch_types=[
            pltpu.VMEM((_NCH, _CHUNK), jnp.int32),
            pltpu.VMEM((_BPW, _CP), jnp.float32),
            pltpu.SemaphoreType.DMA,
        ],
        compiler_params=pltpu.CompilerParams(use_tc_tiling_on_sc=False),
    )
    def gather(cb_hbm, idx_hbm, out_hbm, idx_v, rows_v, sem):
        wid = lax.axis_index("s") * _SC_INFO.num_cores + lax.axis_index("c")
        base = wid * _BPW
        for j in range(_NCH):
            pltpu.sync_copy(idx_hbm.at[pl.ds(base + j * _CHUNK, _CHUNK)],
                            idx_v.at[j])
        copies = [
            pltpu.async_copy(cb_hbm.at[idx_v.at[j]],
                             rows_v.at[pl.ds(j * _CHUNK, _CHUNK)], sem)
            for j in range(_NCH)
        ]
        for cp in copies:
            cp.wait()
        pltpu.sync_copy(rows_v, out_hbm.at[pl.ds(base, _BPW)])

    return gather


_sc_gather = _make_sc_gather()


def kernel(z, codebook):
    zp = jnp.transpose(z, (0, 2, 3, 1))
    zf = zp.reshape(_N, _C)
    zsq = jnp.sum(zp ** 2, axis=3).reshape(_N, 1)
    csq = jnp.sum(codebook ** 2, axis=1).reshape(1, _K)
    idx2d, msum = _dist_argmin(zf, codebook, zsq, csq)
    idx = idx2d.reshape(_N)
    zq = _sc_gather(codebook, idx)
    out = jnp.transpose(zq.reshape(_B, _H, _W, _C), (0, 3, 1, 2))
    m = msum[0] / (_N * _C)
    loss = m + _BETA * m
    indices = idx.reshape(_B, 1, _H, _W)
    return out, loss, indices


# speedup vs baseline: 1.1197x; 1.1197x over previous
import functools

import jax
import jax.numpy as jnp
from jax import lax
from jax.experimental import pallas as pl
from jax.experimental.pallas import tpu as pltpu
from jax.experimental.pallas import tpu_sc as plsc

_B, _C, _H, _W = 8, 64, 32, 32
_K = 8192
_N = _B * _H * _W
_BETA = 0.25

_R = 2048
_KB = _K // 2
_NR = _N // _R


def _dist_argmin_body(zf_ref, cb_ref, zsq_ref, csq_ref,
                      idx_ref, msum_ref, v0_ref, i0_ref, acc_ref):
    i = pl.program_id(0)
    k = pl.program_id(1)
    mm = lax.dot_general(zf_ref[...], cb_ref[...],
                         (((1,), (1,)), ((), ())),
                         preferred_element_type=jnp.float32)
    d = (zsq_ref[...] + csq_ref[...]) - 2.0 * mm
    lmin = jnp.min(d, axis=1, keepdims=True)
    idsf = lax.broadcasted_iota(jnp.int32, (_R, _KB), 1).astype(jnp.float32)
    lidxf = jnp.min(jnp.where(d == lmin, idsf, float(_K)),
                    axis=1, keepdims=True)

    @pl.when(k == 0)
    def _():
        v0_ref[...] = lmin
        i0_ref[...] = lidxf

    @pl.when(k == 1)
    def _():
        v0 = v0_ref[...]
        v0r = v0.astype(jnp.bfloat16).astype(jnp.float32)
        take = lmin < v0r
        idxf = jnp.where(take, lidxf + float(_KB), i0_ref[...])
        idx_ref[...] = idxf.astype(jnp.int32)
        vsel = jnp.where(take, lmin, v0)
        prev = jnp.where(i == 0, 0.0, acc_ref[0])
        acc_ref[0] = prev + jnp.sum(vsel)

        @pl.when(i == _NR - 1)
        def _():
            msum_ref[0] = acc_ref[0]


_dist_argmin = pl.pallas_call(
    _dist_argmin_body,
    grid=(_NR, 2),
    in_specs=[
        pl.BlockSpec((_R, _C), lambda i, k: (i, 0)),
        pl.BlockSpec((_KB, _C), lambda i, k: (k, 0)),
        pl.BlockSpec((_R, 1), lambda i, k: (i, 0)),
        pl.BlockSpec((1, _KB), lambda i, k: (0, k)),
    ],
    out_specs=(
        pl.BlockSpec((_R, 1), lambda i, k: (i, 0)),
        pl.BlockSpec(memory_space=pltpu.SMEM),
    ),
    out_shape=(
        jax.ShapeDtypeStruct((_N, 1), jnp.int32),
        jax.ShapeDtypeStruct((1,), jnp.float32),
    ),
    scratch_shapes=[
        pltpu.VMEM((_R, 1), jnp.float32),
        pltpu.VMEM((_R, 1), jnp.float32),
        pltpu.SMEM((1,), jnp.float32),
    ],
)


_SC_INFO = plsc.get_sparse_core_info()
_NTILES = _SC_INFO.num_cores * _SC_INFO.num_subcores
_BPW = _N // _NTILES
_CHUNK = 128
_NCH = _BPW // _CHUNK
_CP = 128


def _make_sc_gather():
    mesh = plsc.VectorSubcoreMesh(core_axis_name="c", subcore_axis_name="s")

    @functools.partial(
        pl.kernel, mesh=mesh,
        out_type=jax.ShapeDtypeStruct((_N, _CP), jnp.float32),
        scratch_types=[
            pltpu.VMEM((_NCH, _CHUNK), jnp.int32),
            pltpu.VMEM((_BPW, _CP), jnp.float32),
            pltpu.SemaphoreType.DMA,
        ],
    )
    def gather(cb_hbm, idx_hbm, out_hbm, idx_v, rows_v, sem):
        wid = lax.axis_index("s") * _SC_INFO.num_cores + lax.axis_index("c")
        base = wid * _BPW
        for j in range(_NCH):
            pltpu.sync_copy(idx_hbm.at[pl.ds(base + j * _CHUNK, _CHUNK)],
                            idx_v.at[j])
        copies = [
            pltpu.async_copy(cb_hbm.at[idx_v.at[j]],
                             rows_v.at[pl.ds(j * _CHUNK, _CHUNK)], sem)
            for j in range(_NCH)
        ]
        for cp in copies:
            cp.wait()
        pltpu.sync_copy(rows_v, out_hbm.at[pl.ds(base, _BPW)])

    return gather


_sc_gather = _make_sc_gather()


def kernel(z, codebook):
    zp = jnp.transpose(z, (0, 2, 3, 1))
    zf = zp.reshape(_N, _C)
    zsq = jnp.sum(zp ** 2, axis=3).reshape(_N, 1)
    csq = jnp.sum(codebook ** 2, axis=1).reshape(1, _K)
    idx2d, msum = _dist_argmin(zf, codebook, zsq, csq)
    idx = idx2d.reshape(_N)
    cb_pad = jnp.pad(codebook, ((0, 0), (0, _CP - _C)))
    zq = _sc_gather(cb_pad, idx)[:, :_C]
    out = jnp.transpose(zq.reshape(_B, _H, _W, _C), (0, 3, 1, 2))
    m = msum[0] / (_N * _C)
    loss = m + _BETA * m
    indices = idx.reshape(_B, 1, _H, _W)
    return out, loss, indices
